# baseline (speedup 1.0000x reference)
import functools

import jax
import jax.numpy as jnp
from jax import lax
from jax.experimental import pallas as pl
from jax.experimental.pallas import tpu as pltpu
from jax.experimental.pallas import tpu_sc as plsc

_H = 512
_W = 512
_L = 19
_B = 4
_NP = _B * _L
_R1SQ = 41 * 41
_R2 = 41.0
_ROWS = 88
_NW = 32


def _tc_body(heat_ref, out_ref):
    b = pl.program_id(0)
    l = pl.program_id(1)

    @pl.when((b == 0) & (l == 0))
    def _init():
        out_ref[0] = 0.0

    def slab(i, acc):
        p = heat_ref[0, 0, pl.ds(i * 8, 8), :]
        a = jnp.abs(p)
        t = jnp.exp2(a * -1.4426950408889634)
        return acc + jnp.maximum(p, 0.0) + jnp.log2(1.0 + t) * 0.6931471805599453

    acc = lax.fori_loop(0, _H // 8, slab, jnp.zeros((8, _W), jnp.float32),
                        unroll=4)
    out_ref[0] += jnp.sum(acc)


def _sc_masked_sums(fmr, idx, colp, rowp):
    mesh = plsc.VectorSubcoreMesh(core_axis_name="c", subcore_axis_name="s")

    @functools.partial(
        pl.kernel,
        mesh=mesh,
        out_type=jax.ShapeDtypeStruct((_NW, 4, 16), jnp.float32),
        scratch_types=[
            pltpu.VMEM((3, 2, _ROWS), jnp.int32),
            pltpu.VMEM((3, 2, _ROWS, 128), jnp.float32),
            pltpu.VMEM((2, 256), jnp.float32),
            pltpu.VMEM((2, _ROWS, 16), jnp.float32),
            pltpu.VMEM((4, 16), jnp.float32),
            pltpu.SemaphoreType.DMA,
        ],
    )
    def k(fmr_hbm, idx_hbm, colp_hbm, rowp_hbm, out_hbm,
          idx_v, data_v, colp_v, rowp_v, acc_v, sem):
        wid = lax.axis_index("s") * 2 + lax.axis_index("c")
        zero = jnp.zeros((16,), jnp.float32)
        one = jnp.ones((16,), jnp.float32)
        for a in range(4):
            acc_v[a] = zero
        for t in range(3):
            pair = wid + t * _NW

            @pl.when(pair < _NP)
            def _do():
                pltpu.sync_copy(idx_hbm.at[pair], idx_v)
                pltpu.sync_copy(colp_hbm.at[pair], colp_v)
                pltpu.sync_copy(rowp_hbm.at[pair], rowp_v)
                cps = [
                    pltpu.async_copy(
                        fmr_hbm.at[idx_v.at[ch, q]], data_v.at[ch, q], sem)
                    for ch in range(3) for q in range(2)
                ]
                for cp in cps:
                    cp.wait()

                def row_body(r, carry):
                    ax, ay, ap, ac = carry
                    thresh = rowp_v[0, r]
                    gty = rowp_v[1, r]
                    for c in range(16):
                        q, off = c // 8, (c % 8) * 16
                        m = colp_v[0, pl.ds(c * 16, 16)] <= thresh
                        hp = data_v[0, q, r, pl.ds(off, 16)]
                        fx = data_v[1, q, r, pl.ds(off, 16)]
                        fy = data_v[2, q, r, pl.ds(off, 16)]
                        gtx = colp_v[1, pl.ds(c * 16, 16)]
                        ax = ax + jnp.where(m, jnp.abs(fx - gtx), 0.0)
                        ay = ay + jnp.where(m, jnp.abs(fy - gty), 0.0)
                        ap = ap + jnp.where(m, hp, 0.0)
                        ac = ac + jnp.where(m, one, 0.0)
                    return ax, ay, ap, ac

                ax, ay, ap, ac = lax.fori_loop(
                    0, _ROWS, row_body, (zero, zero, zero, zero))
                acc_v[0] = acc_v[0] + ax
                acc_v[1] = acc_v[1] + ay
                acc_v[2] = acc_v[2] + ap
                acc_v[3] = acc_v[3] + ac

        pltpu.sync_copy(acc_v, out_hbm.at[wid])

    return k(fmr, idx, colp, rowp)


def kernel(feature_maps, landmarks):
    h, w = feature_maps.shape[2], feature_maps.shape[3]
    nl = feature_maps.shape[1] // 3
    X = jnp.clip((landmarks[:, :, 0] * (h - 1)).astype(jnp.int32), 0, h - 1)
    Y = jnp.clip((landmarks[:, :, 1] * (w - 1)).astype(jnp.int32), 0, w - 1)

    tc_partials = pl.pallas_call(
        _tc_body,
        grid=(_B, _L),
        in_specs=[
            pl.BlockSpec((1, 1, _H, _W), lambda b, l: (b, l, 0, 0)),
        ],
        out_specs=pl.BlockSpec(memory_space=pltpu.SMEM),
        out_shape=jax.ShapeDtypeStruct((1,), jnp.float32),
    )(feature_maps)

    Xf = X.reshape(_NP)
    Yf = Y.reshape(_NP)
    l_idx = jnp.tile(jnp.arange(_L, dtype=jnp.int32), _B)
    b_idx = jnp.repeat(jnp.arange(_B, dtype=jnp.int32), _L)
    r0 = jnp.clip(Xf - 41, 0, _H - _ROWS)
    q0 = jnp.clip((Yf - 41) // 128, 0, 2)

    ch = jnp.stack([l_idx, nl + l_idx, 2 * nl + l_idx], axis=1)
    rows = r0[:, None] + jnp.arange(_ROWS, dtype=jnp.int32)
    base = (b_idx[:, None] * (3 * nl) + ch) * _H
    idx = ((base[:, :, None] + rows[:, None, :]) * 4)[:, :, None, :] \
        + (q0[:, None] + jnp.arange(2, dtype=jnp.int32))[:, None, :, None]
    idx = idx.astype(jnp.int32)

    jcols = q0[:, None] * 128 + jnp.arange(256, dtype=jnp.int32)
    djc = (jcols - Yf[:, None]).astype(jnp.float32)
    colp = jnp.stack([djc * djc, -djc / _R2], axis=1)
    dir_ = (rows - Xf[:, None]).astype(jnp.float32)
    rowp = jnp.stack([_R1SQ - dir_ * dir_, -dir_ / _R2], axis=1)
    rowp = jnp.broadcast_to(rowp[..., None], (_NP, 2, _ROWS, 16))

    fmr = feature_maps.reshape(-1, 128)
    sc_out = _sc_masked_sums(fmr, idx, colp, rowp)

    ox_sum = jnp.sum(sc_out[:, 0, :])
    oy_sum = jnp.sum(sc_out[:, 1, :])
    pg_sum = jnp.sum(sc_out[:, 2, :])
    mask_sum = jnp.sum(sc_out[:, 3, :])

    bce = (tc_partials[0] - pg_sum) / jnp.float32(_B * nl * h * w)
    denom = jnp.maximum(mask_sum, 1.0)
    return 2.0 * bce + (ox_sum + oy_sum) / denom


# speedup vs baseline: 1.0202x; 1.0202x over previous
import functools

import jax
import jax.numpy as jnp
from jax import lax
from jax.experimental import pallas as pl
from jax.experimental.pallas import tpu as pltpu
from jax.experimental.pallas import tpu_sc as plsc

_H = 512
_W = 512
_L = 19
_B = 4
_NP = _B * _L
_R1SQ = 41 * 41
_R2 = 41.0
_ROWS = 88
_NW = 32


def _tc_body(heat_ref, out_ref):
    b = pl.program_id(0)
    l = pl.program_id(1)

    @pl.when((b == 0) & (l == 0))
    def _init():
        out_ref[0] = 0.0

    p = heat_ref[0, 0]
    a = jnp.abs(p)
    t = jnp.exp2(a * -1.4426950408889634)
    s = jnp.maximum(p, 0.0) + jnp.log2(1.0 + t) * 0.6931471805599453
    out_ref[0] += jnp.sum(s)


def _sc_masked_sums(fmr, idx, colp, rowp, pparm):
    mesh = plsc.VectorSubcoreMesh(core_axis_name="c", subcore_axis_name="s")

    @functools.partial(
        pl.kernel,
        mesh=mesh,
        out_type=jax.ShapeDtypeStruct((_NW, 4, 16), jnp.float32),
        scratch_types=[
            pltpu.VMEM((3, 2, _ROWS), jnp.int32),
            pltpu.VMEM((3, _ROWS, 256), jnp.float32),
            pltpu.VMEM((2, 256), jnp.float32),
            pltpu.VMEM((2, _ROWS, 16), jnp.float32),
            pltpu.VMEM((16,), jnp.int32),
            pltpu.VMEM((4, 16), jnp.float32),
            pltpu.SemaphoreType.DMA,
        ],
    )
    def k(fmr_hbm, idx_hbm, colp_hbm, rowp_hbm, pparm_hbm, out_hbm,
          idx_v, data_v, colp_v, rowp_v, pparm_v, acc_v, sem):
        wid = lax.axis_index("s") * 2 + lax.axis_index("c")
        zero = jnp.zeros((16,), jnp.float32)
        one = jnp.ones((16,), jnp.float32)
        for a in range(4):
            acc_v[a] = zero
        for t in range(3):
            pair = wid + t * _NW

            @pl.when(pair < _NP)
            def _do():
                pltpu.sync_copy(idx_hbm.at[pair], idx_v)
                pltpu.sync_copy(colp_hbm.at[pair], colp_v)
                pltpu.sync_copy(rowp_hbm.at[pair], rowp_v)
                pltpu.sync_copy(pparm_hbm.at[pair], pparm_v)
                cps = [
                    pltpu.async_copy(
                        fmr_hbm.at[idx_v.at[ch, q]],
                        data_v.at[ch, slice(None), pl.ds(q * 128, 128)], sem)
                    for ch in range(3) for q in range(2)
                ]
                for cp in cps:
                    cp.wait()
                o = pl.multiple_of(pparm_v[...][0], 16)

                def row_body(r, carry):
                    ax, ay, ap, ac = carry
                    thresh = rowp_v[0, r]
                    gty = rowp_v[1, r]
                    for c in range(7):
                        s = o + c * 16
                        m = colp_v[0, pl.ds(s, 16)] <= thresh
                        hp = data_v[0, r, pl.ds(s, 16)]
                        fx = data_v[1, r, pl.ds(s, 16)]
                        fy = data_v[2, r, pl.ds(s, 16)]
                        gtx = colp_v[1, pl.ds(s, 16)]
                        ax = ax + jnp.where(m, jnp.abs(fx - gtx), 0.0)
                        ay = ay + jnp.where(m, jnp.abs(fy - gty), 0.0)
                        ap = ap + jnp.where(m, hp, 0.0)
                        ac = ac + jnp.where(m, one, 0.0)
                    return ax, ay, ap, ac

                ax, ay, ap, ac = lax.fori_loop(
                    0, _ROWS, row_body, (zero, zero, zero, zero))
                acc_v[0] = acc_v[0] + ax
                acc_v[1] = acc_v[1] + ay
                acc_v[2] = acc_v[2] + ap
                acc_v[3] = acc_v[3] + ac

        pltpu.sync_copy(acc_v, out_hbm.at[wid])

    return k(fmr, idx, colp, rowp, pparm)


def kernel(feature_maps, landmarks):
    h, w = feature_maps.shape[2], feature_maps.shape[3]
    nl = feature_maps.shape[1] // 3
    X = jnp.clip((landmarks[:, :, 0] * (h - 1)).astype(jnp.int32), 0, h - 1)
    Y = jnp.clip((landmarks[:, :, 1] * (w - 1)).astype(jnp.int32), 0, w - 1)

    tc_partials = pl.pallas_call(
        _tc_body,
        grid=(_B, _L),
        in_specs=[
            pl.BlockSpec((1, 1, _H, _W), lambda b, l: (b, l, 0, 0)),
        ],
        out_specs=pl.BlockSpec(memory_space=pltpu.SMEM),
        out_shape=jax.ShapeDtypeStruct((1,), jnp.float32),
    )(feature_maps)

    Xf = X.reshape(_NP)
    Yf = Y.reshape(_NP)
    l_idx = jnp.tile(jnp.arange(_L, dtype=jnp.int32), _B)
    b_idx = jnp.repeat(jnp.arange(_B, dtype=jnp.int32), _L)
    r0 = jnp.clip(Xf - 41, 0, _H - _ROWS)
    A = jnp.clip(Yf - 41, 0, _W - 96)
    s0 = jnp.clip(A // 128, 0, (_W - 256) // 128)
    off = jnp.clip((A - s0 * 128) // 16 * 16, 0, 144)

    ch = jnp.stack([l_idx, nl + l_idx, 2 * nl + l_idx], axis=1)
    rows = r0[:, None] + jnp.arange(_ROWS, dtype=jnp.int32)
    base = (b_idx[:, None] * (3 * nl) + ch) * _H
    idx = ((base[:, :, None] + rows[:, None, :]) * 4)[:, :, None, :] \
        + (s0[:, None] + jnp.arange(2, dtype=jnp.int32))[:, None, :, None]
    idx = idx.astype(jnp.int32)

    jcols = s0[:, None] * 128 + jnp.arange(256, dtype=jnp.int32)
    djc = (jcols - Yf[:, None]).astype(jnp.float32)
    colp = jnp.stack([djc * djc, -djc / _R2], axis=1)
    dir_ = (rows - Xf[:, None]).astype(jnp.float32)
    rowp = jnp.stack([_R1SQ - dir_ * dir_, -dir_ / _R2], axis=1)
    rowp = jnp.broadcast_to(rowp[..., None], (_NP, 2, _ROWS, 16))
    pparm = jnp.broadcast_to(off.astype(jnp.int32)[:, None], (_NP, 16))

    fmr = feature_maps.reshape(-1, 128)
    sc_out = _sc_masked_sums(fmr, idx, colp, rowp, pparm)

    ox_sum = jnp.sum(sc_out[:, 0, :])
    oy_sum = jnp.sum(sc_out[:, 1, :])
    pg_sum = jnp.sum(sc_out[:, 2, :])
    mask_sum = jnp.sum(sc_out[:, 3, :])

    bce = (tc_partials[0] - pg_sum) / jnp.float32(_B * nl * h * w)
    denom = jnp.maximum(mask_sum, 1.0)
    return 2.0 * bce + (ox_sum + oy_sum) / denom


# speedup vs baseline: 3.2640x; 3.1995x over previous
import functools

import jax
import jax.numpy as jnp
from jax import lax
from jax.experimental import pallas as pl
from jax.experimental.pallas import tpu as pltpu
from jax.experimental.pallas import tpu_sc as plsc

_H = 512
_W = 512
_L = 19
_B = 4
_NP = _B * _L
_R1SQ = 41 * 41
_R2 = 41.0
_ROWS = 96
_NW = 32


def _tc_body(heat_ref, out_ref):
    b = pl.program_id(0)
    l = pl.program_id(1)

    @pl.when((b == 0) & (l == 0))
    def _init():
        out_ref[0] = 0.0

    p = heat_ref[0, 0]
    a = jnp.abs(p)
    t = jnp.exp2(a * -1.4426950408889634)
    s = jnp.maximum(p, 0.0) + jnp.log2(1.0 + t) * 0.6931471805599453
    out_ref[0] += jnp.sum(s)


def _sc_masked_sums(fm3, colp, rowp, pparm):
    mesh = plsc.VectorSubcoreMesh(core_axis_name="c", subcore_axis_name="s")

    @functools.partial(
        pl.kernel,
        mesh=mesh,
        out_type=jax.ShapeDtypeStruct((_NW, 4, 16), jnp.float32),
        scratch_types=[
            pltpu.VMEM((3, _ROWS, 256), jnp.float32),
            pltpu.VMEM((2, 256), jnp.float32),
            pltpu.VMEM((2, _ROWS, 16), jnp.float32),
            pltpu.VMEM((16,), jnp.int32),
            pltpu.VMEM((4, 16), jnp.float32),
            pltpu.SemaphoreType.DMA,
        ],
    )
    def k(fm3_hbm, colp_hbm, rowp_hbm, pparm_hbm, out_hbm,
          data_v, colp_v, rowp_v, pparm_v, acc_v, sem):
        wid = lax.axis_index("s") * 2 + lax.axis_index("c")
        zero = jnp.zeros((16,), jnp.float32)
        one = jnp.ones((16,), jnp.float32)
        for a in range(4):
            acc_v[a] = zero
        for t in range(3):
            pair = wid + t * _NW

            @pl.when(pair < _NP)
            def _do():
                pltpu.sync_copy(colp_hbm.at[pair], colp_v)
                pltpu.sync_copy(rowp_hbm.at[pair], rowp_v)
                pltpu.sync_copy(pparm_hbm.at[pair], pparm_v)
                pv = pparm_v[...]
                r0 = pl.multiple_of(pv[0], 8)
                c0 = pl.multiple_of(pv[1], 128)
                o = pl.multiple_of(pv[5], 16)
                cps = [
                    pltpu.async_copy(
                        fm3_hbm.at[pv[2 + ci], pl.ds(r0, _ROWS),
                                   pl.ds(c0, 256)],
                        data_v.at[ci], sem)
                    for ci in range(3)
                ]
                for cp in cps:
                    cp.wait()

                def row_body(r, carry):
                    ax, ay, ap, ac = carry
                    thresh = rowp_v[0, r]
                    gty = rowp_v[1, r]
                    for c in range(7):
                        s = o + c * 16
                        m = colp_v[0, pl.ds(s, 16)] <= thresh
                        hp = data_v[0, r, pl.ds(s, 16)]
                        fx = data_v[1, r, pl.ds(s, 16)]
                        fy = data_v[2, r, pl.ds(s, 16)]
                        gtx = colp_v[1, pl.ds(s, 16)]
                        ax = ax + jnp.where(m, jnp.abs(fx - gtx), 0.0)
                        ay = ay + jnp.where(m, jnp.abs(fy - gty), 0.0)
                        ap = ap + jnp.where(m, hp, 0.0)
                        ac = ac + jnp.where(m, one, 0.0)
                    return ax, ay, ap, ac

                ax, ay, ap, ac = lax.fori_loop(
                    0, _ROWS, row_body, (zero, zero, zero, zero))
                acc_v[0] = acc_v[0] + ax
                acc_v[1] = acc_v[1] + ay
                acc_v[2] = acc_v[2] + ap
                acc_v[3] = acc_v[3] + ac

        pltpu.sync_copy(acc_v, out_hbm.at[wid])

    return k(fm3, colp, rowp, pparm)


def kernel(feature_maps, landmarks):
    h, w = feature_maps.shape[2], feature_maps.shape[3]
    nl = feature_maps.shape[1] // 3
    X = jnp.clip((landmarks[:, :, 0] * (h - 1)).astype(jnp.int32), 0, h - 1)
    Y = jnp.clip((landmarks[:, :, 1] * (w - 1)).astype(jnp.int32), 0, w - 1)

    tc_partials = pl.pallas_call(
        _tc_body,
        grid=(_B, _L),
        in_specs=[
            pl.BlockSpec((1, 1, _H, _W), lambda b, l: (b, l, 0, 0)),
        ],
        out_specs=pl.BlockSpec(memory_space=pltpu.SMEM),
        out_shape=jax.ShapeDtypeStruct((1,), jnp.float32),
    )(feature_maps)

    Xf = X.reshape(_NP)
    Yf = Y.reshape(_NP)
    l_idx = jnp.tile(jnp.arange(_L, dtype=jnp.int32), _B)
    b_idx = jnp.repeat(jnp.arange(_B, dtype=jnp.int32), _L)
    r0 = jnp.clip((Xf - 41) // 8 * 8, 0, _H - _ROWS)
    A = jnp.clip(Yf - 41, 0, _W - 96)
    s0 = jnp.clip(A // 128, 0, (_W - 256) // 128)
    off = jnp.clip((A - s0 * 128) // 16 * 16, 0, 144)

    ch = jnp.stack([l_idx, nl + l_idx, 2 * nl + l_idx], axis=1)
    chan = b_idx[:, None] * (3 * nl) + ch
    rows = r0[:, None] + jnp.arange(_ROWS, dtype=jnp.int32)

    jcols = s0[:, None] * 128 + jnp.arange(256, dtype=jnp.int32)
    djc = (jcols - Yf[:, None]).astype(jnp.float32)
    colp = jnp.stack([djc * djc, -djc / _R2], axis=1)
    dir_ = (rows - Xf[:, None]).astype(jnp.float32)
    rowp = jnp.stack([_R1SQ - dir_ * dir_, -dir_ / _R2], axis=1)
    rowp = jnp.broadcast_to(rowp[..., None], (_NP, 2, _ROWS, 16))
    pparm = jnp.concatenate([
        r0[:, None], s0[:, None] * 128, chan, off[:, None],
        jnp.zeros((_NP, 10), jnp.int32)], axis=1).astype(jnp.int32)

    fm3 = feature_maps.reshape(_B * 3 * nl, _H, _W)
    sc_out = _sc_masked_sums(fm3, colp, rowp, pparm)

    ox_sum = jnp.sum(sc_out[:, 0, :])
    oy_sum = jnp.sum(sc_out[:, 1, :])
    pg_sum = jnp.sum(sc_out[:, 2, :])
    mask_sum = jnp.sum(sc_out[:, 3, :])

    bce = (tc_partials[0] - pg_sum) / jnp.float32(_B * nl * h * w)
    denom = jnp.maximum(mask_sum, 1.0)
    return 2.0 * bce + (ox_sum + oy_sum) / denom


# speedup vs baseline: 5.2104x; 1.5963x over previous
import functools

import jax
import jax.numpy as jnp
from jax import lax
from jax.experimental import pallas as pl
from jax.experimental.pallas import tpu as pltpu
from jax.experimental.pallas import tpu_sc as plsc

_H = 512
_W = 512
_L = 19
_B = 4
_NP = _B * _L
_R1SQ = 41 * 41
_R2 = 41.0
_ROWS = 96
_NW = 32


def _tc_body(heat_ref, out_ref):
    @pl.when(pl.program_id(0) == 0)
    def _init():
        out_ref[0] = 0.0

    p = heat_ref[0]
    a = jnp.abs(p)
    t = jnp.exp2(a * -1.4426950408889634)
    s = jnp.maximum(p, 0.0) + jnp.log2(1.0 + t) * 0.6931471805599453
    out_ref[0] += jnp.sum(s)


def _sc_masked_sums(fm3, pparm):
    mesh = plsc.VectorSubcoreMesh(core_axis_name="c", subcore_axis_name="s")

    @functools.partial(
        pl.kernel,
        mesh=mesh,
        out_type=jax.ShapeDtypeStruct((_NW, 4, 16), jnp.float32),
        scratch_types=[
            pltpu.VMEM((3, _ROWS, 256), jnp.float32),
            pltpu.VMEM((16,), jnp.int32),
            pltpu.VMEM((4, 16), jnp.float32),
            pltpu.SemaphoreType.DMA,
        ],
    )
    def k(fm3_hbm, pparm_hbm, out_hbm, data_v, pparm_v, acc_v, sem):
        wid = lax.axis_index("s") * 2 + lax.axis_index("c")
        zero = jnp.zeros((16,), jnp.float32)
        one = jnp.ones((16,), jnp.float32)
        iota = lax.iota(jnp.int32, 16)
        for a in range(4):
            acc_v[a] = zero
        for t in range(3):
            pair = wid + t * _NW

            @pl.when(pair < _NP)
            def _do():
                pltpu.sync_copy(pparm_hbm.at[pair], pparm_v)
                pv = pparm_v[...]
                r0 = pl.multiple_of(pv[0], 8)
                c0 = pl.multiple_of(pv[1], 128)
                o = pl.multiple_of(pv[5], 16)
                X = pv[6]
                Y = pv[7]
                cps = [
                    pltpu.async_copy(
                        fm3_hbm.at[pv[2 + ci], pl.ds(r0, _ROWS),
                                   pl.ds(c0, 256)],
                        data_v.at[ci], sem)
                    for ci in range(3)
                ]
                cbase = c0 + o - Y
                cols = []
                for c in range(7):
                    djv = (iota + (cbase + c * 16)).astype(jnp.float32)
                    cols.append((djv * djv, djv * (-1.0 / _R2)))
                for cp in cps:
                    cp.wait()

                def row_body(r, carry):
                    ax, ay, ap, ac = carry
                    div = jnp.full((16,), r0 + r - X, jnp.int32)
                    divf = div.astype(jnp.float32)
                    thresh = _R1SQ - divf * divf
                    gty = divf * (-1.0 / _R2)
                    for c in range(7):
                        djsq, gtx = cols[c]
                        s = o + c * 16
                        m = djsq <= thresh
                        hp = data_v[0, r, pl.ds(s, 16)]
                        fx = data_v[1, r, pl.ds(s, 16)]
                        fy = data_v[2, r, pl.ds(s, 16)]
                        ax = ax + jnp.where(m, jnp.abs(fx - gtx), 0.0)
                        ay = ay + jnp.where(m, jnp.abs(fy - gty), 0.0)
                        ap = ap + jnp.where(m, hp, 0.0)
                        ac = ac + jnp.where(m, one, 0.0)
                    return ax, ay, ap, ac

                ax, ay, ap, ac = lax.fori_loop(
                    0, _ROWS, row_body, (zero, zero, zero, zero))
                acc_v[0] = acc_v[0] + ax
                acc_v[1] = acc_v[1] + ay
                acc_v[2] = acc_v[2] + ap
                acc_v[3] = acc_v[3] + ac

        pltpu.sync_copy(acc_v, out_hbm.at[wid])

    return k(fm3, pparm)


def kernel(feature_maps, landmarks):
    h, w = feature_maps.shape[2], feature_maps.shape[3]
    nl = feature_maps.shape[1] // 3
    X = jnp.clip((landmarks[:, :, 0] * (h - 1)).astype(jnp.int32), 0, h - 1)
    Y = jnp.clip((landmarks[:, :, 1] * (w - 1)).astype(jnp.int32), 0, w - 1)

    tc_partials = pl.pallas_call(
        _tc_body,
        grid=(_B,),
        in_specs=[
            pl.BlockSpec((1, _L, _H, _W), lambda b: (b, 0, 0, 0)),
        ],
        out_specs=pl.BlockSpec(memory_space=pltpu.SMEM),
        out_shape=jax.ShapeDtypeStruct((1,), jnp.float32),
    )(feature_maps)

    Xf = X.reshape(_NP)
    Yf = Y.reshape(_NP)
    l_idx = jnp.tile(jnp.arange(_L, dtype=jnp.int32), _B)
    b_idx = jnp.repeat(jnp.arange(_B, dtype=jnp.int32), _L)
    r0 = jnp.clip((Xf - 41) // 8 * 8, 0, _H - _ROWS)
    A = jnp.clip(Yf - 41, 0, _W - 96)
    s0 = jnp.clip(A // 128, 0, (_W - 256) // 128)
    off = jnp.clip((A - s0 * 128) // 16 * 16, 0, 144)

    ch = jnp.stack([l_idx, nl + l_idx, 2 * nl + l_idx], axis=1)
    chan = b_idx[:, None] * (3 * nl) + ch
    pparm = jnp.concatenate([
        r0[:, None], s0[:, None] * 128, chan, off[:, None],
        Xf[:, None], Yf[:, None],
        jnp.zeros((_NP, 8), jnp.int32)], axis=1).astype(jnp.int32)

    fm3 = feature_maps.reshape(_B * 3 * nl, _H, _W)
    sc_out = _sc_masked_sums(fm3, pparm)

    ox_sum = jnp.sum(sc_out[:, 0, :])
    oy_sum = jnp.sum(sc_out[:, 1, :])
    pg_sum = jnp.sum(sc_out[:, 2, :])
    mask_sum = jnp.sum(sc_out[:, 3, :])

    bce = (tc_partials[0] - pg_sum) / jnp.float32(_B * nl * h * w)
    denom = jnp.maximum(mask_sum, 1.0)
    return 2.0 * bce + (ox_sum + oy_sum) / denom


# speedup vs baseline: 5.3706x; 1.0307x over previous
import functools

import jax
import jax.numpy as jnp
from jax import lax
from jax.experimental import pallas as pl
from jax.experimental.pallas import tpu as pltpu
from jax.experimental.pallas import tpu_sc as plsc

_H = 512
_W = 512
_L = 19
_B = 4
_NP = _B * _L
_R1SQ = 41 * 41
_R2 = 41.0
_ROWS = 96
_NW = 32


def _tc_body(heat_ref, out_ref):
    @pl.when((pl.program_id(0) == 0) & (pl.program_id(1) == 0))
    def _init():
        out_ref[0] = 0.0

    q = heat_ref[0] * 1.4426950408889634
    qi = lax.bitcast_convert_type(q, jnp.int32)
    n = lax.bitcast_convert_type(qi | jnp.int32(-2147483648), jnp.float32)
    s = jnp.maximum(q, 0.0) + jnp.log2(1.0 + jnp.exp2(n))
    out_ref[0] += jnp.sum(s)


def _sc_masked_sums(fm3, pparm):
    mesh = plsc.VectorSubcoreMesh(core_axis_name="c", subcore_axis_name="s")

    @functools.partial(
        pl.kernel,
        mesh=mesh,
        out_type=jax.ShapeDtypeStruct((_NW, 4, 16), jnp.float32),
        scratch_types=[
            pltpu.VMEM((3, _ROWS, 256), jnp.float32),
            pltpu.VMEM((16,), jnp.int32),
            pltpu.VMEM((4, 16), jnp.float32),
            pltpu.SemaphoreType.DMA,
        ],
    )
    def k(fm3_hbm, pparm_hbm, out_hbm, data_v, pparm_v, acc_v, sem):
        wid = lax.axis_index("s") * 2 + lax.axis_index("c")
        zero = jnp.zeros((16,), jnp.float32)
        one = jnp.ones((16,), jnp.float32)
        iota = lax.iota(jnp.int32, 16)
        for a in range(4):
            acc_v[a] = zero

        def pair_body(t, _):
            pair = wid + t * _NW

            @pl.when(pair < _NP)
            def _do():
                pltpu.sync_copy(pparm_hbm.at[pair], pparm_v)
                pv = pparm_v[...]
                r0 = pl.multiple_of(pv[0], 8)
                c0 = pl.multiple_of(pv[1], 128)
                o = pl.multiple_of(pv[5], 16)
                X = pv[6]
                Y = pv[7]
                cps = [
                    pltpu.async_copy(
                        fm3_hbm.at[pv[2 + ci], pl.ds(r0, _ROWS),
                                   pl.ds(c0, 256)],
                        data_v.at[ci], sem)
                    for ci in range(3)
                ]
                cbase = c0 + o - Y
                cols = []
                for c in range(7):
                    djv = (iota + (cbase + c * 16)).astype(jnp.float32)
                    cols.append((djv * djv, djv * (-1.0 / _R2)))
                for cp in cps:
                    cp.wait()

                def row_body(r, carry):
                    ax, ay, ap, ac = carry
                    div = jnp.full((16,), r0 + r - X, jnp.int32)
                    divf = div.astype(jnp.float32)
                    thresh = _R1SQ - divf * divf
                    gty = divf * (-1.0 / _R2)
                    for c in range(7):
                        djsq, gtx = cols[c]
                        s = o + c * 16
                        m = djsq <= thresh
                        hp = data_v[0, r, pl.ds(s, 16)]
                        fx = data_v[1, r, pl.ds(s, 16)]
                        fy = data_v[2, r, pl.ds(s, 16)]
                        ax = ax + jnp.where(m, jnp.abs(fx - gtx), 0.0)
                        ay = ay + jnp.where(m, jnp.abs(fy - gty), 0.0)
                        ap = ap + jnp.where(m, hp, 0.0)
                        ac = ac + jnp.where(m, one, 0.0)
                    return ax, ay, ap, ac

                ax, ay, ap, ac = lax.fori_loop(
                    0, _ROWS, row_body, (zero, zero, zero, zero))
                acc_v[0] = acc_v[0] + ax
                acc_v[1] = acc_v[1] + ay
                acc_v[2] = acc_v[2] + ap
                acc_v[3] = acc_v[3] + ac

            return 0

        lax.fori_loop(0, 3, pair_body, 0)
        pltpu.sync_copy(acc_v, out_hbm.at[wid])

    return k(fm3, pparm)


def kernel(feature_maps, landmarks):
    h, w = feature_maps.shape[2], feature_maps.shape[3]
    nl = feature_maps.shape[1] // 3
    X = jnp.clip((landmarks[:, :, 0] * (h - 1)).astype(jnp.int32), 0, h - 1)
    Y = jnp.clip((landmarks[:, :, 1] * (w - 1)).astype(jnp.int32), 0, w - 1)

    tc_partials = pl.pallas_call(
        _tc_body,
        grid=(_B, 2),
        in_specs=[
            pl.BlockSpec((1, _L, _H // 2, _W), lambda b, i: (b, 0, i, 0)),
        ],
        out_specs=pl.BlockSpec(memory_space=pltpu.SMEM),
        out_shape=jax.ShapeDtypeStruct((1,), jnp.float32),
    )(feature_maps)

    Xf = X.reshape(_NP)
    Yf = Y.reshape(_NP)
    l_idx = jnp.tile(jnp.arange(_L, dtype=jnp.int32), _B)
    b_idx = jnp.repeat(jnp.arange(_B, dtype=jnp.int32), _L)
    r0 = jnp.clip((Xf - 41) // 8 * 8, 0, _H - _ROWS)
    A = jnp.clip(Yf - 41, 0, _W - 96)
    s0 = jnp.clip(A // 128, 0, (_W - 256) // 128)
    off = jnp.clip((A - s0 * 128) // 16 * 16, 0, 144)

    ch = jnp.stack([l_idx, nl + l_idx, 2 * nl + l_idx], axis=1)
    chan = b_idx[:, None] * (3 * nl) + ch
    pparm = jnp.concatenate([
        r0[:, None], s0[:, None] * 128, chan, off[:, None],
        Xf[:, None], Yf[:, None],
        jnp.zeros((_NP, 8), jnp.int32)], axis=1).astype(jnp.int32)

    fm3 = feature_maps.reshape(_B * 3 * nl, _H, _W)
    sc_out = _sc_masked_sums(fm3, pparm)

    ox_sum = jnp.sum(sc_out[:, 0, :])
    oy_sum = jnp.sum(sc_out[:, 1, :])
    pg_sum = jnp.sum(sc_out[:, 2, :])
    mask_sum = jnp.sum(sc_out[:, 3, :])

    bce_dense = tc_partials[0] * 0.6931471805599453
    bce = (bce_dense - pg_sum) / jnp.float32(_B * nl * h * w)
    denom = jnp.maximum(mask_sum, 1.0)
    return 2.0 * bce + (ox_sum + oy_sum) / denom
